# baseline (speedup 1.0000x reference)
import jax
import jax.numpy as jnp
from jax.experimental import pallas as pl
from jax.experimental.pallas import tpu as pltpu

N = 40
TXT_FEAT_LEN = 1386
HIDDEN = 4096
HEADS = 8
CODE_LEN = 64
NEG_SLOPE = 0.2

A_TILE = 512
N_A = HIDDEN // A_TILE
BAND = 1024
N_BAND = HIDDEN // BAND
B_TILE = 512
N_B = (HEADS * HIDDEN) // B_TILE
N_BSTEP = N_B // 2
SUBS = HIDDEN // B_TILE


def _mask_degrees(G):
    Mf = (G != -1.5).astype(jnp.float32)
    D = jnp.sum(Mf, axis=1)
    B = jnp.sum(Mf, axis=0)
    Dinv = jnp.where(D > 0.0, 1.0 / D, 0.0)
    Binv = jnp.where(B > 0.0, 1.0 / B, 0.0)
    return Mf, B, Dinv, Binv


def _mm(a, b):
    return jax.lax.dot_general(a, b, (((1,), (0,)), ((), ())),
                               preferred_element_type=jnp.float32)


def _mtm(a, b):
    return jax.lax.dot_general(a, b, (((0,), (0,)), ((), ())),
                               preferred_element_type=jnp.float32)


def _fused_kernel(x_ref, w1_ref, g_ref, b1_ref, w2a_ref, w2b_ref, att1_ref,
                  att2_ref, b2_ref, w3_ref, b3_ref, feat_ref, hid_ref,
                  code_ref, fc_s, xw_s):
    t = pl.program_id(0)

    @pl.when(t < N_A)
    def _phase_a():
        Mf, B, Dinv, Binv = _mask_degrees(g_ref[...])
        xw1 = _mm(x_ref[...].astype(jnp.bfloat16),
                  w1_ref[...].astype(jnp.bfloat16))
        ef = Binv[:, None] * _mtm(Mf, xw1)
        feat = jnp.maximum(Dinv[:, None] * _mm(Mf, ef) + b1_ref[...], 0.0)
        feat_ref[...] = feat
        hattr = _mtm(Mf, feat) / B[:, None]
        fc = jnp.concatenate([feat, hattr], axis=0)
        fc_s[t // 2, :, pl.ds((t % 2) * A_TILE, A_TILE)] = fc

    @pl.when((t >= N_A) & (t < N_A + N_BSTEP))
    def _phase_b():
        c = t - N_A
        acc_a = jnp.zeros((2 * N, B_TILE), jnp.float32)
        acc_b = jnp.zeros((2 * N, B_TILE), jnp.float32)
        for k in range(N_BAND):
            acc_a += _mm(fc_s[k], w2a_ref[k * BAND:(k + 1) * BAND, :])
            acc_b += _mm(fc_s[k], w2b_ref[k * BAND:(k + 1) * BAND, :])
        xw_s[2 * c] = acc_a
        xw_s[2 * c + 1] = acc_b

    @pl.when(t == N_A + N_BSTEP)
    def _phase_c():
        Mb = g_ref[...] != -1.5
        Mf, B, Dinv, Binv = _mask_degrees(g_ref[...])

        As = []
        for h in range(HEADS):
            ax = jnp.zeros((N,), jnp.float32)
            ae = jnp.zeros((N,), jnp.float32)
            for s in range(SUBS):
                blk = xw_s[SUBS * h + s]
                a1 = att1_ref[h, s * B_TILE:(s + 1) * B_TILE]
                a2 = att2_ref[h, s * B_TILE:(s + 1) * B_TILE]
                ax += jnp.sum(blk[:N] * a1, axis=1)
                ae += jnp.sum(blk[N:] * a2, axis=1)
            al = ax[:, None] + ae[None, :]
            al = jnp.where(al >= 0.0, al, NEG_SLOPE * al)
            amax = jnp.max(jnp.where(Mb, al, -jnp.inf), axis=1)
            amax = jnp.where(amax > -jnp.inf, amax, 0.0)
            aexp = jnp.where(Mb, jnp.exp(al - amax[:, None]), 0.0)
            asum = jnp.sum(aexp, axis=1)
            As.append(aexp / (asum[:, None] + 1e-16))

        xw3 = jnp.zeros((N, CODE_LEN), jnp.float32)
        for s in range(SUBS):
            acc = jnp.zeros((N, B_TILE), jnp.float32)
            for h in range(HEADS):
                xh = xw_s[SUBS * h + s][:N]
                ef = Binv[:, None] * _mtm(As[h], xh)
                acc = acc + Dinv[:, None] * _mm(As[h], ef)
            featsub = fc_s[s * B_TILE // BAND][:N,
                           (s * B_TILE) % BAND:(s * B_TILE) % BAND + B_TILE]
            hcs = (featsub + acc * (1.0 / HEADS)
                   + b2_ref[0, s * B_TILE:(s + 1) * B_TILE])
            xw3 = xw3 + _mm(hcs, w3_ref[s * B_TILE:(s + 1) * B_TILE, :])
        ef3 = Binv[:, None] * _mtm(Mf, xw3)
        hid = Dinv[:, None] * _mm(Mf, ef3) + b3_ref[...]
        hid_ref[...] = hid
        code_ref[...] = jnp.tanh(hid)


def kernel(x, G, W1, b1, W2, att, b2, W3, b3):
    att1 = att[0, :, :HIDDEN]
    att2 = att[0, :, HIDDEN:]
    a_last = N_A - 1
    b_last = N_BSTEP - 1
    grid = N_A + N_BSTEP + 1
    feat, hid, code = pl.pallas_call(
        _fused_kernel,
        grid=(grid,),
        in_specs=[
            pl.BlockSpec((N, TXT_FEAT_LEN), lambda t: (0, 0)),
            pl.BlockSpec((TXT_FEAT_LEN, A_TILE),
                         lambda t: (0, jnp.minimum(t, a_last))),
            pl.BlockSpec((N, N), lambda t: (0, 0)),
            pl.BlockSpec((1, A_TILE), lambda t: (0, jnp.minimum(t, a_last))),
            pl.BlockSpec((HIDDEN, B_TILE),
                         lambda t: (0, 2 * jnp.clip(t - N_A, 0, b_last))),
            pl.BlockSpec((HIDDEN, B_TILE),
                         lambda t: (0, 2 * jnp.clip(t - N_A, 0, b_last) + 1)),
            pl.BlockSpec((HEADS, HIDDEN), lambda t: (0, 0)),
            pl.BlockSpec((HEADS, HIDDEN), lambda t: (0, 0)),
            pl.BlockSpec((1, HIDDEN), lambda t: (0, 0)),
            pl.BlockSpec((HIDDEN, CODE_LEN), lambda t: (0, 0)),
            pl.BlockSpec((1, CODE_LEN), lambda t: (0, 0)),
        ],
        out_specs=[
            pl.BlockSpec((N, A_TILE), lambda t: (0, jnp.minimum(t, a_last))),
            pl.BlockSpec((N, CODE_LEN), lambda t: (0, 0)),
            pl.BlockSpec((N, CODE_LEN), lambda t: (0, 0)),
        ],
        out_shape=[
            jax.ShapeDtypeStruct((N, HIDDEN), jnp.float32),
            jax.ShapeDtypeStruct((N, CODE_LEN), jnp.float32),
            jax.ShapeDtypeStruct((N, CODE_LEN), jnp.float32),
        ],
        scratch_shapes=[
            pltpu.VMEM((N_BAND, 2 * N, BAND), jnp.float32),
            pltpu.VMEM((N_B, 2 * N, B_TILE), jnp.float32),
        ],
        compiler_params=pltpu.CompilerParams(
            dimension_semantics=("arbitrary",)),
    )(x, W1, G, b1.reshape(1, HIDDEN), W2, W2, att1, att2,
      b2.reshape(1, HIDDEN), W3, b3.reshape(1, CODE_LEN))
    return (feat, hid, code)


# speedup vs baseline: 1.0444x; 1.0444x over previous
import jax
import jax.numpy as jnp
from jax.experimental import pallas as pl
from jax.experimental.pallas import tpu as pltpu

N = 40
TXT_FEAT_LEN = 1386
HIDDEN = 4096
HEADS = 8
CODE_LEN = 64
NEG_SLOPE = 0.2

A_TILE = 512
N_A = HIDDEN // A_TILE
BAND = 1024
N_BAND = HIDDEN // BAND
B_TILE = 1024
N_B = (HEADS * HIDDEN) // B_TILE
SUBS = HIDDEN // B_TILE


def _mask_degrees(G):
    Mf = (G != -1.5).astype(jnp.float32)
    D = jnp.sum(Mf, axis=1)
    B = jnp.sum(Mf, axis=0)
    Dinv = jnp.where(D > 0.0, 1.0 / D, 0.0)
    Binv = jnp.where(B > 0.0, 1.0 / B, 0.0)
    return Mf, B, Dinv, Binv


def _mm(a, b):
    return jax.lax.dot_general(a, b, (((1,), (0,)), ((), ())),
                               preferred_element_type=jnp.float32)


def _mtm(a, b):
    return jax.lax.dot_general(a, b, (((0,), (0,)), ((), ())),
                               preferred_element_type=jnp.float32)


def _fused_kernel(x_ref, w1_ref, g_ref, b1_ref, w2_ref, att1_ref,
                  att2_ref, b2_ref, w3_ref, b3_ref, feat_ref, hid_ref,
                  code_ref, fc_s, xw_s, hacc_s):
    t = pl.program_id(0)

    @pl.when(t < N_A)
    def _phase_a():
        Mf, B, Dinv, Binv = _mask_degrees(g_ref[...])
        xw1 = _mm(x_ref[...], w1_ref[...])
        ef = Binv[:, None] * _mtm(Mf, xw1)
        feat = jnp.maximum(Dinv[:, None] * _mm(Mf, ef) + b1_ref[...], 0.0)
        feat_ref[...] = feat
        hattr = _mtm(Mf, feat) / B[:, None]
        fc = jnp.concatenate([feat, hattr], axis=0)
        fc_s[t // 2, :, pl.ds((t % 2) * A_TILE, A_TILE)] = fc

    @pl.when((t >= N_A) & (t < N_A + N_B))
    def _phase_b():
        c = t - N_A
        acc = jnp.zeros((2 * N, B_TILE), jnp.float32)
        for k in range(N_BAND):
            acc += _mm(fc_s[k], w2_ref[k * BAND:(k + 1) * BAND, :])
        xw_s[c] = acc

        @pl.when(c % SUBS == SUBS - 1)
        def _head_work():
            h = c // SUBS
            Mb = g_ref[...] != -1.5
            Mf, B, Dinv, Binv = _mask_degrees(g_ref[...])
            ax = jnp.zeros((N,), jnp.float32)
            ae = jnp.zeros((N,), jnp.float32)
            for s in range(SUBS):
                blk = xw_s[c - (SUBS - 1) + s]
                a1 = att1_ref[h, s * B_TILE:(s + 1) * B_TILE]
                a2 = att2_ref[h, s * B_TILE:(s + 1) * B_TILE]
                ax += jnp.sum(blk[:N] * a1, axis=1)
                ae += jnp.sum(blk[N:] * a2, axis=1)
            al = ax[:, None] + ae[None, :]
            al = jnp.where(al >= 0.0, al, NEG_SLOPE * al)
            amax = jnp.max(jnp.where(Mb, al, -jnp.inf), axis=1)
            amax = jnp.where(amax > -jnp.inf, amax, 0.0)
            aexp = jnp.where(Mb, jnp.exp(al - amax[:, None]), 0.0)
            asum = jnp.sum(aexp, axis=1)
            A = aexp / (asum[:, None] + 1e-16)
            for s in range(SUBS):
                xh = xw_s[c - (SUBS - 1) + s][:N]
                ef = Binv[:, None] * _mtm(A, xh)
                oh = Dinv[:, None] * _mm(A, ef)

                @pl.when(h == 0)
                def _hinit():
                    hacc_s[s] = oh

                @pl.when(h > 0)
                def _haccum():
                    hacc_s[s] = hacc_s[s] + oh

    @pl.when(t == N_A + N_B)
    def _phase_c():
        Mf, B, Dinv, Binv = _mask_degrees(g_ref[...])
        xw3 = jnp.zeros((N, CODE_LEN), jnp.float32)
        for s in range(SUBS):
            featsub = fc_s[s][:N]
            hcs = (featsub + hacc_s[s] * (1.0 / HEADS)
                   + b2_ref[0, s * B_TILE:(s + 1) * B_TILE])
            xw3 = xw3 + _mm(hcs, w3_ref[s * B_TILE:(s + 1) * B_TILE, :])
        ef3 = Binv[:, None] * _mtm(Mf, xw3)
        hid = Dinv[:, None] * _mm(Mf, ef3) + b3_ref[...]
        hid_ref[...] = hid
        code_ref[...] = jnp.tanh(hid)


def kernel(x, G, W1, b1, W2, att, b2, W3, b3):
    att1 = att[0, :, :HIDDEN]
    att2 = att[0, :, HIDDEN:]
    a_last = N_A - 1
    b_last = N_B - 1
    grid = N_A + N_B + 1
    feat, hid, code = pl.pallas_call(
        _fused_kernel,
        grid=(grid,),
        in_specs=[
            pl.BlockSpec((N, TXT_FEAT_LEN), lambda t: (0, 0)),
            pl.BlockSpec((TXT_FEAT_LEN, A_TILE),
                         lambda t: (0, jnp.minimum(t, a_last))),
            pl.BlockSpec((N, N), lambda t: (0, 0)),
            pl.BlockSpec((1, A_TILE), lambda t: (0, jnp.minimum(t, a_last))),
            pl.BlockSpec((HIDDEN, B_TILE),
                         lambda t: (0, jnp.clip(t - N_A, 0, b_last))),
            pl.BlockSpec((HEADS, HIDDEN), lambda t: (0, 0)),
            pl.BlockSpec((HEADS, HIDDEN), lambda t: (0, 0)),
            pl.BlockSpec((1, HIDDEN), lambda t: (0, 0)),
            pl.BlockSpec((HIDDEN, CODE_LEN), lambda t: (0, 0)),
            pl.BlockSpec((1, CODE_LEN), lambda t: (0, 0)),
        ],
        out_specs=[
            pl.BlockSpec((N, A_TILE), lambda t: (0, jnp.minimum(t, a_last))),
            pl.BlockSpec((N, CODE_LEN), lambda t: (0, 0)),
            pl.BlockSpec((N, CODE_LEN), lambda t: (0, 0)),
        ],
        out_shape=[
            jax.ShapeDtypeStruct((N, HIDDEN), jnp.float32),
            jax.ShapeDtypeStruct((N, CODE_LEN), jnp.float32),
            jax.ShapeDtypeStruct((N, CODE_LEN), jnp.float32),
        ],
        scratch_shapes=[
            pltpu.VMEM((N_BAND, 2 * N, BAND), jnp.float32),
            pltpu.VMEM((N_B, 2 * N, B_TILE), jnp.float32),
            pltpu.VMEM((SUBS, N, B_TILE), jnp.float32),
        ],
        compiler_params=pltpu.CompilerParams(
            dimension_semantics=("arbitrary",)),
    )(x, W1, G, b1.reshape(1, HIDDEN), W2, att1, att2,
      b2.reshape(1, HIDDEN), W3, b3.reshape(1, CODE_LEN))
    return (feat, hid, code)
